# baseline (speedup 1.0000x reference)
import functools

import jax
import jax.numpy as jnp
from jax import lax
from jax.experimental import pallas as pl
from jax.experimental.pallas import tpu as pltpu
from jax.experimental.pallas import tpu_sc as plsc

PAR = 10000
ITEM = 5000
TRAIT = 32
OPT = 5
DIM = 128
HID = 512
B = 4096

K = 128
SUP = 4
NTILE = 16
NCORE = 2

REP = 32
RW = 32 * REP

GRAPHS = (
    ("pt", PAR, TRAIT, 100352, 11040, 1, True),
    ("pu", PAR, ITEM, 160256, 15040, 0, False),
    ("tu", ITEM, TRAIT, 50176, 6048, 1, True),
    ("po", PAR, OPT, 40448, 11040, 1, True),
    ("uo", ITEM, OPT, 20480, 6048, 0, True),
)
ACCN = 10016
TRASH = 10000
DEGN = 15072
DEGPAD = 15071


def _ru32(x):
    return -(-x // 32) * 32


def _f32(c):
    return jnp.full((16,), c, dtype=jnp.float32)


def _newton_rsqrt(d):
    i = lax.bitcast_convert_type(d, jnp.int32)
    i = jnp.full((16,), 0x5F3759DF, dtype=jnp.int32) - lax.shift_right_arithmetic(
        i, jnp.full((16,), 1, dtype=jnp.int32)
    )
    x = lax.bitcast_convert_type(i, jnp.float32)
    for _ in range(3):
        x = x * (_f32(1.5) - _f32(0.5) * d * x * x)
    return x


def _row_loop(vec32, fn, n=32):
    def body(j, _):
        rj = plsc.load_gather(vec32, [jnp.zeros((16,), jnp.int32) + j])
        fn(j, rj)
        return 0

    lax.fori_loop(0, n, body, 0)


def _sc_gcn_body(refs, sid, cid):
    (xs, srcs, dscats, ddegs, os, ys, hs, acc, degr, idxs, idxd, gb, gb2,
     blk_a, blk_b, zb, vec32, vecb, sem, sem2) = refs

    for j in range(16):
        for t in range(8):
            zb[j, pl.ds(t * 16, 16)] = _f32(0.0)

    def strided_trips(total):
        base = total // NTILE
        rem = total % NTILE
        return base + jnp.where(sid < rem, 1, 0)

    def deg_phase(ddeg2, nsup):
        for t in range(K // 16):
            gb[0, pl.ds(t * 16, 16)] = _f32(1.0)
        ones = gb.at[0, pl.ds(0, K)]

        def body(k, _):
            s = sid + k * NTILE
            pltpu.sync_copy(ddeg2.at[pl.ds(s * SUP, SUP)], idxd)
            descs = [
                pltpu.async_copy(ones, degr.at[idxd.at[c]], sem, add=True)
                for c in range(SUP)
            ]
            for d in descs:
                d.wait()
            return 0

        lax.fori_loop(0, strided_trips(nsup), body, 0)

    def edge_phase(src2, dscat2, y, s0, nsup):
        gbufs = (gb, gb2)
        sems = (sem, sem2)

        def body(k, _):
            s = s0 + sid + k * NTILE
            pltpu.sync_copy(src2.at[pl.ds(s * SUP, SUP)], idxs)
            pltpu.sync_copy(dscat2.at[pl.ds(s * SUP, SUP)], idxd)
            pend = pltpu.async_copy(y.at[idxs.at[0]], gbufs[0], sems[0])
            for c in range(SUP):
                pend.wait()
                if c + 1 < SUP:
                    nx = (c + 1) % 2
                    pend = pltpu.async_copy(
                        y.at[idxs.at[c + 1]], gbufs[nx], sems[nx])
                pltpu.sync_copy(gbufs[c % 2], acc.at[idxd.at[c]], add=True)
            return 0

        lax.fori_loop(0, strided_trips(nsup), body, 0)

    def blocks32(cnt32, body_fn):
        def body(i, _):
            ws = (sid + i * NTILE) * 32
            body_fn(ws)
            return 0

        lax.fori_loop(0, strided_trips(cnt32 // 32), body, 0)

    def load_r(gs):
        pltpu.sync_copy(degr.at[pl.ds(gs, 32)], vec32)

    def zero_acc_rows(ws):
        pltpu.sync_copy(zb, acc.at[pl.ds(ws, 16)])
        pltpu.sync_copy(zb, acc.at[pl.ds(ws + 16, 16)])

    def add_blocks(dst_blk, src_blk):
        def body(j, _):
            for t in range(8):
                sl = pl.ds(t * 16, 16)
                dst_blk[j, sl] = dst_blk[j, sl] + src_blk[j, sl]
            return 0

        lax.fori_loop(0, 32, body, 0)

    def run_graph(x, src2, dscat2, ddeg2, o, y, h, na, nb, nsup, n_pad, rep):
        wa, wb = _ru32(na), _ru32(nb)

        vec32[pl.ds(0, 16)] = _f32(0.0)
        vec32[pl.ds(16, 16)] = _f32(0.0)

        def p0(ws):
            pltpu.sync_copy(vec32, degr.at[pl.ds(ws, 32)])

        blocks32(n_pad, p0)
        plsc.subcore_barrier()

        deg_phase(ddeg2, 2 * nsup)
        plsc.subcore_barrier()

        if rep:
            @pl.when(sid == 0)
            def _():
                pltpu.sync_copy(degr.at[pl.ds(na, 32)], vec32)

                def body(kr, _):
                    pltpu.sync_copy(degr.at[pl.ds(na + 32 * kr, 32)], vecb)
                    for hf in (0, 16):
                        vec32[pl.ds(hf, 16)] = (vec32[pl.ds(hf, 16)]
                                                + vecb[pl.ds(hf, 16)])
                    return 0

                lax.fori_loop(1, REP, body, 0)
                pltpu.sync_copy(vec32, degr.at[pl.ds(na, 32)])

            plsc.subcore_barrier()

        def p2(ws):
            load_r(ws)
            for hf in (0, 16):
                v = vec32[pl.ds(hf, 16)]
                vec32[pl.ds(hf, 16)] = _newton_rsqrt(
                    jnp.maximum(v, _f32(1.0)))
            pltpu.sync_copy(vec32, degr.at[pl.ds(ws, 32)])
            pltpu.sync_copy(x.at[pl.ds(ws, 32)], blk_a)

            def scale(j, rj):
                for t in range(8):
                    sl = pl.ds(t * 16, 16)
                    blk_a[j, sl] = blk_a[j, sl] * rj

            _row_loop(vec32, scale)
            pltpu.sync_copy(blk_a, y.at[pl.ds(ws, 32)])

        blocks32(n_pad, p2)
        blocks32(wa, zero_acc_rows)
        plsc.subcore_barrier()

        def fan_out_y():
            @pl.when(sid == 0)
            def _():
                pltpu.sync_copy(y.at[pl.ds(na, 32)], blk_a)

                def body(kr, _):
                    pltpu.sync_copy(blk_a, y.at[pl.ds(na + 32 * kr, 32)])
                    return 0

                lax.fori_loop(1, REP, body, 0)

        if rep:
            fan_out_y()
            plsc.subcore_barrier()

        def flush_hy(go, cnt32):
            def f(ws):
                gs = go + ws
                load_r(gs)
                pltpu.sync_copy(acc.at[pl.ds(ws, 32)], blk_a)

                def scale(j, rj):
                    for t in range(8):
                        sl = pl.ds(t * 16, 16)
                        h1 = blk_a[j, sl] * rj
                        blk_a[j, sl] = h1
                        blk_b[j, sl] = h1 * rj

                _row_loop(vec32, scale)
                pltpu.sync_copy(blk_a, h.at[pl.ds(gs, 32)])
                pltpu.sync_copy(blk_b, y.at[pl.ds(gs, 32)])
                zero_acc_rows(ws)

            blocks32(cnt32, f)

        def flush_o(go, cnt32):
            def f(ws):
                gs = go + ws
                load_r(gs)
                pltpu.sync_copy(acc.at[pl.ds(ws, 32)], blk_a)
                pltpu.sync_copy(x.at[pl.ds(gs, 32)], blk_b)

                def scale(j, rj):
                    for t in range(8):
                        sl = pl.ds(t * 16, 16)
                        blk_b[j, sl] = blk_b[j, sl] + blk_a[j, sl] * rj

                _row_loop(vec32, scale)
                pltpu.sync_copy(h.at[pl.ds(gs, 32)], blk_a)

                def fin(j, _):
                    third = _f32(1.0 / 3.0)
                    for t in range(8):
                        sl = pl.ds(t * 16, 16)
                        blk_b[j, sl] = (blk_b[j, sl] + blk_a[j, sl]) * third
                    return 0

                lax.fori_loop(0, 32, fin, 0)
                pltpu.sync_copy(blk_b, o.at[pl.ds(gs, 32)])
                zero_acc_rows(ws)

            blocks32(cnt32, f)

        def reduce_replicas():
            load_r(na)
            pltpu.sync_copy(acc.at[pl.ds(0, 32)], blk_a)

            def body(kr, _):
                pltpu.sync_copy(acc.at[pl.ds(32 * kr, 32)], blk_b)
                add_blocks(blk_a, blk_b)
                return 0

            lax.fori_loop(1, REP, body, 0)

        def flush_hy_rep():
            @pl.when(sid == 0)
            def _():
                reduce_replicas()

                def scale(j, rj):
                    for t in range(8):
                        sl = pl.ds(t * 16, 16)
                        h1 = blk_a[j, sl] * rj
                        blk_a[j, sl] = h1
                        blk_b[j, sl] = h1 * rj

                _row_loop(vec32, scale)
                pltpu.sync_copy(blk_a, h.at[pl.ds(na, 32)])

                def body(kr, _):
                    pltpu.sync_copy(blk_b, y.at[pl.ds(na + 32 * kr, 32)])
                    return 0

                lax.fori_loop(0, REP, body, 0)

            plsc.subcore_barrier()
            blocks32(RW, zero_acc_rows)

        def flush_o_rep():
            @pl.when(sid == 0)
            def _():
                reduce_replicas()

                def scale(j, rj):
                    for t in range(8):
                        sl = pl.ds(t * 16, 16)
                        blk_a[j, sl] = blk_a[j, sl] * rj

                _row_loop(vec32, scale)
                pltpu.sync_copy(x.at[pl.ds(na, 32)], blk_b)
                add_blocks(blk_b, blk_a)
                pltpu.sync_copy(h.at[pl.ds(na, 32)], blk_a)

                def fin(j, _):
                    third = _f32(1.0 / 3.0)
                    for t in range(8):
                        sl = pl.ds(t * 16, 16)
                        blk_b[j, sl] = (blk_b[j, sl] + blk_a[j, sl]) * third
                    return 0

                lax.fori_loop(0, 32, fin, 0)
                pltpu.sync_copy(blk_b, o.at[pl.ds(na, 32)])

        for flush, flush_rep in ((flush_hy, flush_hy_rep),
                                 (flush_o, flush_o_rep)):
            edge_phase(src2, dscat2, y, nsup, nsup)
            plsc.subcore_barrier()
            flush(0, wa)
            plsc.subcore_barrier()
            edge_phase(src2, dscat2, y, 0, nsup)
            plsc.subcore_barrier()
            if rep:
                flush_rep()
            else:
                flush(na, wb)
            plsc.subcore_barrier()

    for gi, (_, na, nb, eh, n_pad, core, rep) in enumerate(GRAPHS):
        @pl.when(cid == core)
        def _(gi=gi, na=na, nb=nb, eh=eh, n_pad=n_pad, rep=rep):
            run_graph(xs[gi], srcs[gi], dscats[gi], ddegs[gi], os[gi],
                      ys[gi], hs[gi], na, nb, eh // (K * SUP), n_pad, rep)


def _sc_gcn(xs, srcs, dscats, ddegs):
    n_graphs = len(GRAPHS)

    def body(*refs):
        groups = [refs[i * n_graphs:(i + 1) * n_graphs] for i in range(7)]
        scratch = refs[7 * n_graphs:]
        sid = lax.axis_index("s")
        cid = lax.axis_index("c")
        _sc_gcn_body(tuple(groups) + tuple(scratch), sid, cid)

    out_type = tuple(
        jax.ShapeDtypeStruct((g[4], DIM), jnp.float32) for g in GRAPHS
    ) * 3
    scratch = [
        pltpu.VMEM_SHARED((ACCN, DIM), jnp.float32),
        pltpu.VMEM_SHARED((DEGN,), jnp.float32),
        pltpu.VMEM((SUP, K), jnp.int32),
        pltpu.VMEM((SUP, K), jnp.int32),
        pltpu.VMEM((K, DIM), jnp.float32),
        pltpu.VMEM((K, DIM), jnp.float32),
        pltpu.VMEM((32, DIM), jnp.float32),
        pltpu.VMEM((32, DIM), jnp.float32),
        pltpu.VMEM((16, DIM), jnp.float32),
        pltpu.VMEM((32,), jnp.float32),
        pltpu.VMEM((32,), jnp.float32),
        pltpu.SemaphoreType.DMA,
        pltpu.SemaphoreType.DMA,
    ]
    mesh = plsc.VectorSubcoreMesh(
        core_axis_name="c", subcore_axis_name="s", num_cores=NCORE,
        num_subcores=NTILE,
    )
    fn = pl.kernel(body, out_type=out_type, mesh=mesh,
                   scratch_types=tuple(scratch),
                   compiler_params=pltpu.CompilerParams(
                       needs_layout_passes=False))
    outs = fn(*xs, *srcs, *dscats, *ddegs)
    return outs[0:n_graphs]


def _score_kernel(z_ref, w_ref, b_ref, q_ref, o_ref, *, n, blk):
    i = pl.program_id(0)

    @pl.when(i == 0)
    def _():
        o_ref[...] = jnp.zeros_like(o_ref)

    t = jnp.tanh(
        jax.lax.dot_general(z_ref[...], w_ref[...], (((1,), (0,)), ((), ())),
                            preferred_element_type=jnp.float32)
        + b_ref[...]
    )
    s = jnp.sum(t * q_ref[...])
    slab = i // (n // blk)
    lane = lax.broadcasted_iota(jnp.int32, (1, 128), 1)
    o_ref[...] += jnp.where(lane == slab, s, 0.0)


def _scores(z, w, b, q, n):
    blk = 1000
    grid = z.shape[0] // blk
    out = pl.pallas_call(
        functools.partial(_score_kernel, n=n, blk=blk),
        grid=(grid,),
        in_specs=[
            pl.BlockSpec((blk, DIM), lambda i: (i, 0)),
            pl.BlockSpec((DIM, DIM), lambda i: (0, 0)),
            pl.BlockSpec((1, DIM), lambda i: (0, 0)),
            pl.BlockSpec((1, DIM), lambda i: (0, 0)),
        ],
        out_specs=pl.BlockSpec((1, 128), lambda i: (0, 0)),
        out_shape=jax.ShapeDtypeStruct((1, 128), jnp.float32),
    )(z, w, b.reshape(1, DIM), q.reshape(1, DIM))
    return out[0, :3] / n


def _combine_kernel(z0_ref, z1_ref, z2_ref, beta_ref, w_ref, b_ref, o_ref):
    comb = (beta_ref[0, 0] * z0_ref[...] + beta_ref[0, 1] * z1_ref[...]
            + beta_ref[0, 2] * z2_ref[...])
    o_ref[...] = jax.lax.dot_general(comb, w_ref[...], (((1,), (0,)), ((), ())),
                                     preferred_element_type=jnp.float32) + b_ref[...]


def _combine_project(z0, z1, z2, beta, w_pad, b_pad):
    n = z0.shape[0]
    blk = 1000
    beta_v = jnp.zeros((1, 128), jnp.float32).at[0, :3].set(beta)
    return pl.pallas_call(
        _combine_kernel,
        grid=(n // blk,),
        in_specs=[
            pl.BlockSpec((blk, DIM), lambda i: (i, 0)),
            pl.BlockSpec((blk, DIM), lambda i: (i, 0)),
            pl.BlockSpec((blk, DIM), lambda i: (i, 0)),
            pl.BlockSpec((1, 128), lambda i: (0, 0)),
            pl.BlockSpec((DIM, DIM), lambda i: (0, 0)),
            pl.BlockSpec((1, DIM), lambda i: (0, 0)),
        ],
        out_specs=pl.BlockSpec((blk, DIM), lambda i: (i, 0)),
        out_shape=jax.ShapeDtypeStruct((n, DIM), jnp.float32),
    )(z0, z1, z2, beta_v, w_pad, b_pad)


def _sc_batch_gather(p_tab, u_tab, d_vec, par_id, item_id):
    nw = NCORE * NTILE
    bpw = B // nw

    def body(p_hbm, u_hbm, d_hbm, pid_hbm, iid_hbm, pg, ug, dg,
             idxv, rowbuf, dbuf, sem):
        wid = lax.axis_index("s") * NCORE + lax.axis_index("c")
        base = wid * bpw
        pltpu.sync_copy(pid_hbm.at[pl.ds(base, bpw)], idxv)
        pltpu.async_copy(p_hbm.at[idxv], rowbuf, sem).wait()
        pltpu.sync_copy(rowbuf, pg.at[pl.ds(base, bpw)])
        pltpu.sync_copy(iid_hbm.at[pl.ds(base, bpw)], idxv)
        pltpu.async_copy(u_hbm.at[idxv], rowbuf, sem).wait()
        pltpu.sync_copy(rowbuf, ug.at[pl.ds(base, bpw)])
        pltpu.async_copy(d_hbm.at[idxv], dbuf, sem).wait()
        pltpu.sync_copy(dbuf, dg.at[pl.ds(base, bpw)])

    mesh = plsc.VectorSubcoreMesh(
        core_axis_name="c", subcore_axis_name="s", num_cores=NCORE,
        num_subcores=NTILE,
    )
    fn = pl.kernel(
        body,
        out_type=(
            jax.ShapeDtypeStruct((B, DIM), jnp.float32),
            jax.ShapeDtypeStruct((B, DIM), jnp.float32),
            jax.ShapeDtypeStruct((B,), jnp.float32),
        ),
        mesh=mesh,
        scratch_types=(
            pltpu.VMEM((bpw,), jnp.int32),
            pltpu.VMEM((bpw, DIM), jnp.float32),
            pltpu.VMEM((bpw,), jnp.float32),
            pltpu.SemaphoreType.DMA,
        ),
        compiler_params=pltpu.CompilerParams(needs_layout_passes=False),
    )
    return fn(p_tab, u_tab, d_vec, par_id, item_id)


def _mlp_kernel(pg_ref, ug_ref, dg_ref, tr_ref, w1_ref, b1_ref, w2_ref,
                b2_ref, w3_ref, b3_ref, o_ref):
    disc = jax.nn.sigmoid(dg_ref[...])
    x = (jax.nn.sigmoid(pg_ref[...]) - jax.nn.sigmoid(ug_ref[...]))
    x = x * tr_ref[...] * disc
    h = jax.nn.sigmoid(
        jax.lax.dot_general(x, jnp.abs(w1_ref[...]), (((1,), (1,)), ((), ())),
                            preferred_element_type=jnp.float32) + b1_ref[...])
    h = jax.nn.sigmoid(
        jax.lax.dot_general(h, jnp.abs(w2_ref[...]), (((1,), (1,)), ((), ())),
                            preferred_element_type=jnp.float32) + b2_ref[...])
    o_ref[...] = jax.lax.dot_general(
        h, jnp.abs(w3_ref[...]), (((1,), (1,)), ((), ())),
        preferred_element_type=jnp.float32) + b3_ref[...]


def _mlp(pg, ug, dg, traits_pad, w1p, b1, w2, b2, w3p, b3p):
    blk = 1024
    return pl.pallas_call(
        _mlp_kernel,
        grid=(B // blk,),
        in_specs=[
            pl.BlockSpec((blk, DIM), lambda i: (i, 0)),
            pl.BlockSpec((blk, DIM), lambda i: (i, 0)),
            pl.BlockSpec((blk, 1), lambda i: (i, 0)),
            pl.BlockSpec((blk, DIM), lambda i: (i, 0)),
            pl.BlockSpec((HID, DIM), lambda i: (0, 0)),
            pl.BlockSpec((1, HID), lambda i: (0, 0)),
            pl.BlockSpec((HID // 2, HID), lambda i: (0, 0)),
            pl.BlockSpec((1, HID // 2), lambda i: (0, 0)),
            pl.BlockSpec((128, HID // 2), lambda i: (0, 0)),
            pl.BlockSpec((1, 128), lambda i: (0, 0)),
        ],
        out_specs=pl.BlockSpec((blk, 128), lambda i: (i, 0)),
        out_shape=jax.ShapeDtypeStruct((B, 128), jnp.float32),
    )(pg, ug, dg, traits_pad, w1p, b1, w2, b2, w3p, b3p)


def _pad_rows(x, n_pad):
    return jnp.pad(x, ((0, n_pad - x.shape[0]), (0, 0)))


def kernel(params, par_id, item_id, traits, e_pt, e_pu, e_tu, e_po, e_uo, e_oo):
    par = params["par_emb"]
    item = params["item_emb"]
    trait = params["trait_emb"]
    opt = params["option_emb"]

    xs = (
        _pad_rows(jnp.concatenate([par, trait]), GRAPHS[0][4]),
        _pad_rows(jnp.concatenate([par, item]), GRAPHS[1][4]),
        _pad_rows(jnp.concatenate([item, trait]), GRAPHS[2][4]),
        _pad_rows(jnp.concatenate([par, opt]), GRAPHS[3][4]),
        _pad_rows(jnp.concatenate([item, opt]), GRAPHS[4][4]),
    )
    edges = (e_pt, e_pu, e_tu, e_po, e_uo)

    def _prep_edges(e, g):
        _, na, nb, eh, n_pad, _, rep = g
        e2 = e.shape[1] // 2
        pad = eh - e2
        roff = (32 * (jnp.arange(e2, dtype=jnp.int32) % REP)) if rep else 0

        def halves(v1, v2, pad_val):
            return jnp.concatenate([
                v1, jnp.full((pad,), pad_val, jnp.int32),
                v2, jnp.full((pad,), pad_val, jnp.int32),
            ]).reshape(2 * eh // K, K)

        src2 = halves(e[0, :e2], e[0, e2:] + roff, n_pad - 1)
        dscat2 = halves(e[1, :e2] - na + roff, e[1, e2:], TRASH)
        ddeg2 = halves(e[1, :e2] + roff, e[1, e2:], DEGPAD)
        return src2, dscat2, ddeg2

    prepped = tuple(_prep_edges(e, g) for e, g in zip(edges, GRAPHS))
    srcs = tuple(p[0] for p in prepped)
    dscats = tuple(p[1] for p in prepped)
    ddegs = tuple(p[2] for p in prepped)

    o_pt, o_pu, o_tu, o_po, o_uo = _sc_gcn(xs, srcs, dscats, ddegs)

    zu0, zu1, zu2 = o_pu[PAR:PAR + ITEM], o_tu[:ITEM], o_uo[:ITEM]
    zp0, zp1, zp2 = o_pu[:PAR], o_pt[:PAR], o_po[:PAR]

    pU = params["attn_U"]
    pP = params["attn_P"]
    wsU = _scores(jnp.concatenate([zu0, zu1, zu2]), pU["W"], pU["b"], pU["q"], ITEM)
    wsP = _scores(jnp.concatenate([zp0, zp1, zp2]), pP["W"], pP["b"], pP["q"], PAR)
    betaU = jax.nn.softmax(wsU)
    betaP = jax.nn.softmax(wsP)

    ti_wp = jnp.pad(params["ti_W"].T, ((0, 0), (0, DIM - TRAIT)))
    ti_bp = jnp.pad(params["ti_b"], (0, DIM - TRAIT)).reshape(1, DIM)
    tp_wp = jnp.pad(params["tp_W"].T, ((0, 0), (0, DIM - TRAIT)))
    tp_bp = jnp.pad(params["tp_b"], (0, DIM - TRAIT)).reshape(1, DIM)

    u_proj = _combine_project(zu0, zu1, zu2, betaU, ti_wp, ti_bp)
    p_proj = _combine_project(zp0, zp1, zp2, betaP, tp_wp, tp_bp)

    pg, ug, dg = _sc_batch_gather(p_proj, u_proj, params["disc_emb"][:, 0],
                                  par_id, item_id)

    traits_pad = jnp.pad(traits, ((0, 0), (0, DIM - TRAIT)))
    w1p = jnp.pad(params["p1_W"], ((0, 0), (0, DIM - TRAIT)))
    b1 = params["p1_b"].reshape(1, HID)
    w2 = params["p2_W"]
    b2 = params["p2_b"].reshape(1, HID // 2)
    w3p = jnp.pad(params["p3_W"], ((0, 128 - OPT), (0, 0)))
    b3p = jnp.pad(params["p3_b"], (0, 128 - OPT)).reshape(1, 128)

    out = _mlp(pg, ug, dg.reshape(B, 1), traits_pad, w1p, b1, w2, b2, w3p, b3p)
    return out[:, :OPT]


# speedup vs baseline: 1.0207x; 1.0207x over previous
import functools

import jax
import jax.numpy as jnp
from jax import lax
from jax.experimental import pallas as pl
from jax.experimental.pallas import tpu as pltpu
from jax.experimental.pallas import tpu_sc as plsc

PAR = 10000
ITEM = 5000
TRAIT = 32
OPT = 5
DIM = 128
HID = 512
B = 4096

K = 128
SUP = 4
NTILE = 16
NCORE = 2

REP = 16
RW = 32 * REP

GRAPHS = (
    ("pt", PAR, TRAIT, 100352, 10528, 1, True),
    ("pu", PAR, ITEM, 160256, 15040, 0, False),
    ("tu", ITEM, TRAIT, 50176, 5536, 1, True),
    ("po", PAR, OPT, 40448, 10528, 1, True),
    ("uo", ITEM, OPT, 20480, 5536, 0, True),
)
ACCN = 10016
TRASH = 10000
DEGN = 15072
DEGPAD = 15071


def _ru32(x):
    return -(-x // 32) * 32


def _f32(c):
    return jnp.full((16,), c, dtype=jnp.float32)


def _newton_rsqrt(d):
    i = lax.bitcast_convert_type(d, jnp.int32)
    i = jnp.full((16,), 0x5F3759DF, dtype=jnp.int32) - lax.shift_right_arithmetic(
        i, jnp.full((16,), 1, dtype=jnp.int32)
    )
    x = lax.bitcast_convert_type(i, jnp.float32)
    for _ in range(3):
        x = x * (_f32(1.5) - _f32(0.5) * d * x * x)
    return x


def _row_loop(vec32, fn, n=32):
    def body(j, _):
        rj = plsc.load_gather(vec32, [jnp.zeros((16,), jnp.int32) + j])
        fn(j, rj)
        return 0

    lax.fori_loop(0, n, body, 0)


def _sc_gcn_body(refs, sid, cid):
    (xs, srcs, dscats, ddegs, os, ys, hs, acc, degr, idxs, idxd, gb, gb2,
     blk_a, blk_b, zb, vec32, vecb, sem, sem2) = refs

    for j in range(16):
        for t in range(8):
            zb[j, pl.ds(t * 16, 16)] = _f32(0.0)

    def strided_trips(total):
        base = total // NTILE
        rem = total % NTILE
        return base + jnp.where(sid < rem, 1, 0)

    def deg_phase(ddeg2, nsup):
        for t in range(K // 16):
            gb[0, pl.ds(t * 16, 16)] = _f32(1.0)
        ones = gb.at[0, pl.ds(0, K)]

        def body(k, _):
            s = sid + k * NTILE
            pltpu.sync_copy(ddeg2.at[pl.ds(s * SUP, SUP)], idxd)
            descs = [
                pltpu.async_copy(ones, degr.at[idxd.at[c]], sem, add=True)
                for c in range(SUP)
            ]
            for d in descs:
                d.wait()
            return 0

        lax.fori_loop(0, strided_trips(nsup), body, 0)

    def edge_phase(src2, dscat2, y, s0, nsup):
        gbufs = (gb, gb2)
        sems = (sem, sem2)

        def body(k, _):
            s = s0 + sid + k * NTILE
            pltpu.sync_copy(src2.at[pl.ds(s * SUP, SUP)], idxs)
            pltpu.sync_copy(dscat2.at[pl.ds(s * SUP, SUP)], idxd)
            pend = pltpu.async_copy(y.at[idxs.at[0]], gbufs[0], sems[0])
            for c in range(SUP):
                pend.wait()
                if c + 1 < SUP:
                    nx = (c + 1) % 2
                    pend = pltpu.async_copy(
                        y.at[idxs.at[c + 1]], gbufs[nx], sems[nx])
                pltpu.sync_copy(gbufs[c % 2], acc.at[idxd.at[c]], add=True)
            return 0

        lax.fori_loop(0, strided_trips(nsup), body, 0)

    def blocks32(cnt32, body_fn):
        def body(i, _):
            ws = (sid + i * NTILE) * 32
            body_fn(ws)
            return 0

        lax.fori_loop(0, strided_trips(cnt32 // 32), body, 0)

    def load_r(gs):
        pltpu.sync_copy(degr.at[pl.ds(gs, 32)], vec32)

    def zero_acc_rows(ws):
        pltpu.sync_copy(zb, acc.at[pl.ds(ws, 16)])
        pltpu.sync_copy(zb, acc.at[pl.ds(ws + 16, 16)])

    def add_blocks(dst_blk, src_blk):
        def body(j, _):
            for t in range(8):
                sl = pl.ds(t * 16, 16)
                dst_blk[j, sl] = dst_blk[j, sl] + src_blk[j, sl]
            return 0

        lax.fori_loop(0, 32, body, 0)

    def run_graph(x, src2, dscat2, ddeg2, o, y, h, na, nb, nsup, n_pad, rep):
        wa, wb = _ru32(na), _ru32(nb)

        vec32[pl.ds(0, 16)] = _f32(0.0)
        vec32[pl.ds(16, 16)] = _f32(0.0)

        def p0(ws):
            pltpu.sync_copy(vec32, degr.at[pl.ds(ws, 32)])

        blocks32(n_pad, p0)
        plsc.subcore_barrier()

        deg_phase(ddeg2, 2 * nsup)
        plsc.subcore_barrier()

        if rep:
            @pl.when(sid == 0)
            def _():
                pltpu.sync_copy(degr.at[pl.ds(na, 32)], vec32)

                def body(kr, _):
                    pltpu.sync_copy(degr.at[pl.ds(na + 32 * kr, 32)], vecb)
                    for hf in (0, 16):
                        vec32[pl.ds(hf, 16)] = (vec32[pl.ds(hf, 16)]
                                                + vecb[pl.ds(hf, 16)])
                    return 0

                lax.fori_loop(1, REP, body, 0)
                pltpu.sync_copy(vec32, degr.at[pl.ds(na, 32)])

            plsc.subcore_barrier()

        def p2(ws):
            load_r(ws)
            for hf in (0, 16):
                v = vec32[pl.ds(hf, 16)]
                vec32[pl.ds(hf, 16)] = _newton_rsqrt(
                    jnp.maximum(v, _f32(1.0)))
            pltpu.sync_copy(vec32, degr.at[pl.ds(ws, 32)])
            pltpu.sync_copy(x.at[pl.ds(ws, 32)], blk_a)

            def scale(j, rj):
                for t in range(8):
                    sl = pl.ds(t * 16, 16)
                    blk_a[j, sl] = blk_a[j, sl] * rj

            _row_loop(vec32, scale)
            pltpu.sync_copy(blk_a, y.at[pl.ds(ws, 32)])

        blocks32(n_pad, p2)
        blocks32(wa, zero_acc_rows)
        plsc.subcore_barrier()

        def fan_out_y():
            @pl.when(sid == 0)
            def _():
                pltpu.sync_copy(y.at[pl.ds(na, 32)], blk_a)

                def body(kr, _):
                    pltpu.sync_copy(blk_a, y.at[pl.ds(na + 32 * kr, 32)])
                    return 0

                lax.fori_loop(1, REP, body, 0)

        if rep:
            fan_out_y()
            plsc.subcore_barrier()

        def flush_hy(go, cnt32):
            def f(ws):
                gs = go + ws
                load_r(gs)
                pltpu.sync_copy(acc.at[pl.ds(ws, 32)], blk_a)

                def scale(j, rj):
                    for t in range(8):
                        sl = pl.ds(t * 16, 16)
                        h1 = blk_a[j, sl] * rj
                        blk_a[j, sl] = h1
                        blk_b[j, sl] = h1 * rj

                _row_loop(vec32, scale)
                pltpu.sync_copy(blk_a, h.at[pl.ds(gs, 32)])
                pltpu.sync_copy(blk_b, y.at[pl.ds(gs, 32)])
                zero_acc_rows(ws)

            blocks32(cnt32, f)

        def flush_o(go, cnt32):
            def f(ws):
                gs = go + ws
                load_r(gs)
                pltpu.sync_copy(acc.at[pl.ds(ws, 32)], blk_a)
                pltpu.sync_copy(x.at[pl.ds(gs, 32)], blk_b)

                def scale(j, rj):
                    for t in range(8):
                        sl = pl.ds(t * 16, 16)
                        blk_b[j, sl] = blk_b[j, sl] + blk_a[j, sl] * rj

                _row_loop(vec32, scale)
                pltpu.sync_copy(h.at[pl.ds(gs, 32)], blk_a)

                def fin(j, _):
                    third = _f32(1.0 / 3.0)
                    for t in range(8):
                        sl = pl.ds(t * 16, 16)
                        blk_b[j, sl] = (blk_b[j, sl] + blk_a[j, sl]) * third
                    return 0

                lax.fori_loop(0, 32, fin, 0)
                pltpu.sync_copy(blk_b, o.at[pl.ds(gs, 32)])
                zero_acc_rows(ws)

            blocks32(cnt32, f)

        def reduce_replicas():
            load_r(na)
            pltpu.sync_copy(acc.at[pl.ds(0, 32)], blk_a)

            def body(kr, _):
                pltpu.sync_copy(acc.at[pl.ds(32 * kr, 32)], blk_b)
                add_blocks(blk_a, blk_b)
                return 0

            lax.fori_loop(1, REP, body, 0)

        def flush_hy_rep():
            @pl.when(sid == 0)
            def _():
                reduce_replicas()

                def scale(j, rj):
                    for t in range(8):
                        sl = pl.ds(t * 16, 16)
                        h1 = blk_a[j, sl] * rj
                        blk_a[j, sl] = h1
                        blk_b[j, sl] = h1 * rj

                _row_loop(vec32, scale)
                pltpu.sync_copy(blk_a, h.at[pl.ds(na, 32)])

                def body(kr, _):
                    pltpu.sync_copy(blk_b, y.at[pl.ds(na + 32 * kr, 32)])
                    return 0

                lax.fori_loop(0, REP, body, 0)

            plsc.subcore_barrier()
            blocks32(RW, zero_acc_rows)

        def flush_o_rep():
            @pl.when(sid == 0)
            def _():
                reduce_replicas()

                def scale(j, rj):
                    for t in range(8):
                        sl = pl.ds(t * 16, 16)
                        blk_a[j, sl] = blk_a[j, sl] * rj

                _row_loop(vec32, scale)
                pltpu.sync_copy(x.at[pl.ds(na, 32)], blk_b)
                add_blocks(blk_b, blk_a)
                pltpu.sync_copy(h.at[pl.ds(na, 32)], blk_a)

                def fin(j, _):
                    third = _f32(1.0 / 3.0)
                    for t in range(8):
                        sl = pl.ds(t * 16, 16)
                        blk_b[j, sl] = (blk_b[j, sl] + blk_a[j, sl]) * third
                    return 0

                lax.fori_loop(0, 32, fin, 0)
                pltpu.sync_copy(blk_b, o.at[pl.ds(na, 32)])

        for flush, flush_rep in ((flush_hy, flush_hy_rep),
                                 (flush_o, flush_o_rep)):
            edge_phase(src2, dscat2, y, nsup, nsup)
            plsc.subcore_barrier()
            flush(0, wa)
            plsc.subcore_barrier()
            edge_phase(src2, dscat2, y, 0, nsup)
            plsc.subcore_barrier()
            if rep:
                flush_rep()
            else:
                flush(na, wb)
            plsc.subcore_barrier()

    for gi, (_, na, nb, eh, n_pad, core, rep) in enumerate(GRAPHS):
        @pl.when(cid == core)
        def _(gi=gi, na=na, nb=nb, eh=eh, n_pad=n_pad, rep=rep):
            run_graph(xs[gi], srcs[gi], dscats[gi], ddegs[gi], os[gi],
                      ys[gi], hs[gi], na, nb, eh // (K * SUP), n_pad, rep)


def _sc_gcn(xs, srcs, dscats, ddegs):
    n_graphs = len(GRAPHS)

    def body(*refs):
        groups = [refs[i * n_graphs:(i + 1) * n_graphs] for i in range(7)]
        scratch = refs[7 * n_graphs:]
        sid = lax.axis_index("s")
        cid = lax.axis_index("c")
        _sc_gcn_body(tuple(groups) + tuple(scratch), sid, cid)

    out_type = tuple(
        jax.ShapeDtypeStruct((g[4], DIM), jnp.float32) for g in GRAPHS
    ) * 3
    scratch = [
        pltpu.VMEM_SHARED((ACCN, DIM), jnp.float32),
        pltpu.VMEM_SHARED((DEGN,), jnp.float32),
        pltpu.VMEM((SUP, K), jnp.int32),
        pltpu.VMEM((SUP, K), jnp.int32),
        pltpu.VMEM((K, DIM), jnp.float32),
        pltpu.VMEM((K, DIM), jnp.float32),
        pltpu.VMEM((32, DIM), jnp.float32),
        pltpu.VMEM((32, DIM), jnp.float32),
        pltpu.VMEM((16, DIM), jnp.float32),
        pltpu.VMEM((32,), jnp.float32),
        pltpu.VMEM((32,), jnp.float32),
        pltpu.SemaphoreType.DMA,
        pltpu.SemaphoreType.DMA,
    ]
    mesh = plsc.VectorSubcoreMesh(
        core_axis_name="c", subcore_axis_name="s", num_cores=NCORE,
        num_subcores=NTILE,
    )
    fn = pl.kernel(body, out_type=out_type, mesh=mesh,
                   scratch_types=tuple(scratch),
                   compiler_params=pltpu.CompilerParams(
                       needs_layout_passes=False))
    outs = fn(*xs, *srcs, *dscats, *ddegs)
    return outs[0:n_graphs]


def _score_kernel(z_ref, w_ref, b_ref, q_ref, o_ref, *, n, blk):
    i = pl.program_id(0)

    @pl.when(i == 0)
    def _():
        o_ref[...] = jnp.zeros_like(o_ref)

    t = jnp.tanh(
        jax.lax.dot_general(z_ref[...], w_ref[...], (((1,), (0,)), ((), ())),
                            preferred_element_type=jnp.float32)
        + b_ref[...]
    )
    s = jnp.sum(t * q_ref[...])
    slab = i // (n // blk)
    lane = lax.broadcasted_iota(jnp.int32, (1, 128), 1)
    o_ref[...] += jnp.where(lane == slab, s, 0.0)


def _scores(z, w, b, q, n):
    blk = 1000
    grid = z.shape[0] // blk
    out = pl.pallas_call(
        functools.partial(_score_kernel, n=n, blk=blk),
        grid=(grid,),
        in_specs=[
            pl.BlockSpec((blk, DIM), lambda i: (i, 0)),
            pl.BlockSpec((DIM, DIM), lambda i: (0, 0)),
            pl.BlockSpec((1, DIM), lambda i: (0, 0)),
            pl.BlockSpec((1, DIM), lambda i: (0, 0)),
        ],
        out_specs=pl.BlockSpec((1, 128), lambda i: (0, 0)),
        out_shape=jax.ShapeDtypeStruct((1, 128), jnp.float32),
    )(z, w, b.reshape(1, DIM), q.reshape(1, DIM))
    return out[0, :3] / n


def _combine_kernel(z0_ref, z1_ref, z2_ref, beta_ref, w_ref, b_ref, o_ref):
    comb = (beta_ref[0, 0] * z0_ref[...] + beta_ref[0, 1] * z1_ref[...]
            + beta_ref[0, 2] * z2_ref[...])
    o_ref[...] = jax.lax.dot_general(comb, w_ref[...], (((1,), (0,)), ((), ())),
                                     preferred_element_type=jnp.float32) + b_ref[...]


def _combine_project(z0, z1, z2, beta, w_pad, b_pad):
    n = z0.shape[0]
    blk = 1000
    beta_v = jnp.zeros((1, 128), jnp.float32).at[0, :3].set(beta)
    return pl.pallas_call(
        _combine_kernel,
        grid=(n // blk,),
        in_specs=[
            pl.BlockSpec((blk, DIM), lambda i: (i, 0)),
            pl.BlockSpec((blk, DIM), lambda i: (i, 0)),
            pl.BlockSpec((blk, DIM), lambda i: (i, 0)),
            pl.BlockSpec((1, 128), lambda i: (0, 0)),
            pl.BlockSpec((DIM, DIM), lambda i: (0, 0)),
            pl.BlockSpec((1, DIM), lambda i: (0, 0)),
        ],
        out_specs=pl.BlockSpec((blk, DIM), lambda i: (i, 0)),
        out_shape=jax.ShapeDtypeStruct((n, DIM), jnp.float32),
    )(z0, z1, z2, beta_v, w_pad, b_pad)


def _sc_batch_gather(p_tab, u_tab, d_vec, par_id, item_id):
    nw = NCORE * NTILE
    bpw = B // nw

    def body(p_hbm, u_hbm, d_hbm, pid_hbm, iid_hbm, pg, ug, dg,
             idxv, rowbuf, dbuf, sem):
        wid = lax.axis_index("s") * NCORE + lax.axis_index("c")
        base = wid * bpw
        pltpu.sync_copy(pid_hbm.at[pl.ds(base, bpw)], idxv)
        pltpu.async_copy(p_hbm.at[idxv], rowbuf, sem).wait()
        pltpu.sync_copy(rowbuf, pg.at[pl.ds(base, bpw)])
        pltpu.sync_copy(iid_hbm.at[pl.ds(base, bpw)], idxv)
        pltpu.async_copy(u_hbm.at[idxv], rowbuf, sem).wait()
        pltpu.sync_copy(rowbuf, ug.at[pl.ds(base, bpw)])
        pltpu.async_copy(d_hbm.at[idxv], dbuf, sem).wait()
        pltpu.sync_copy(dbuf, dg.at[pl.ds(base, bpw)])

    mesh = plsc.VectorSubcoreMesh(
        core_axis_name="c", subcore_axis_name="s", num_cores=NCORE,
        num_subcores=NTILE,
    )
    fn = pl.kernel(
        body,
        out_type=(
            jax.ShapeDtypeStruct((B, DIM), jnp.float32),
            jax.ShapeDtypeStruct((B, DIM), jnp.float32),
            jax.ShapeDtypeStruct((B,), jnp.float32),
        ),
        mesh=mesh,
        scratch_types=(
            pltpu.VMEM((bpw,), jnp.int32),
            pltpu.VMEM((bpw, DIM), jnp.float32),
            pltpu.VMEM((bpw,), jnp.float32),
            pltpu.SemaphoreType.DMA,
        ),
        compiler_params=pltpu.CompilerParams(needs_layout_passes=False),
    )
    return fn(p_tab, u_tab, d_vec, par_id, item_id)


def _mlp_kernel(pg_ref, ug_ref, dg_ref, tr_ref, w1_ref, b1_ref, w2_ref,
                b2_ref, w3_ref, b3_ref, o_ref):
    disc = jax.nn.sigmoid(dg_ref[...])
    x = (jax.nn.sigmoid(pg_ref[...]) - jax.nn.sigmoid(ug_ref[...]))
    x = x * tr_ref[...] * disc
    h = jax.nn.sigmoid(
        jax.lax.dot_general(x, jnp.abs(w1_ref[...]), (((1,), (1,)), ((), ())),
                            preferred_element_type=jnp.float32) + b1_ref[...])
    h = jax.nn.sigmoid(
        jax.lax.dot_general(h, jnp.abs(w2_ref[...]), (((1,), (1,)), ((), ())),
                            preferred_element_type=jnp.float32) + b2_ref[...])
    o_ref[...] = jax.lax.dot_general(
        h, jnp.abs(w3_ref[...]), (((1,), (1,)), ((), ())),
        preferred_element_type=jnp.float32) + b3_ref[...]


def _mlp(pg, ug, dg, traits_pad, w1p, b1, w2, b2, w3p, b3p):
    blk = 1024
    return pl.pallas_call(
        _mlp_kernel,
        grid=(B // blk,),
        in_specs=[
            pl.BlockSpec((blk, DIM), lambda i: (i, 0)),
            pl.BlockSpec((blk, DIM), lambda i: (i, 0)),
            pl.BlockSpec((blk, 1), lambda i: (i, 0)),
            pl.BlockSpec((blk, DIM), lambda i: (i, 0)),
            pl.BlockSpec((HID, DIM), lambda i: (0, 0)),
            pl.BlockSpec((1, HID), lambda i: (0, 0)),
            pl.BlockSpec((HID // 2, HID), lambda i: (0, 0)),
            pl.BlockSpec((1, HID // 2), lambda i: (0, 0)),
            pl.BlockSpec((128, HID // 2), lambda i: (0, 0)),
            pl.BlockSpec((1, 128), lambda i: (0, 0)),
        ],
        out_specs=pl.BlockSpec((blk, 128), lambda i: (i, 0)),
        out_shape=jax.ShapeDtypeStruct((B, 128), jnp.float32),
    )(pg, ug, dg, traits_pad, w1p, b1, w2, b2, w3p, b3p)


def _pad_rows(x, n_pad):
    return jnp.pad(x, ((0, n_pad - x.shape[0]), (0, 0)))


def kernel(params, par_id, item_id, traits, e_pt, e_pu, e_tu, e_po, e_uo, e_oo):
    par = params["par_emb"]
    item = params["item_emb"]
    trait = params["trait_emb"]
    opt = params["option_emb"]

    xs = (
        _pad_rows(jnp.concatenate([par, trait]), GRAPHS[0][4]),
        _pad_rows(jnp.concatenate([par, item]), GRAPHS[1][4]),
        _pad_rows(jnp.concatenate([item, trait]), GRAPHS[2][4]),
        _pad_rows(jnp.concatenate([par, opt]), GRAPHS[3][4]),
        _pad_rows(jnp.concatenate([item, opt]), GRAPHS[4][4]),
    )
    edges = (e_pt, e_pu, e_tu, e_po, e_uo)

    def _prep_edges(e, g):
        _, na, nb, eh, n_pad, _, rep = g
        e2 = e.shape[1] // 2
        pad = eh - e2
        roff = (32 * (jnp.arange(e2, dtype=jnp.int32) % REP)) if rep else 0

        def halves(v1, v2, pad_val):
            return jnp.concatenate([
                v1, jnp.full((pad,), pad_val, jnp.int32),
                v2, jnp.full((pad,), pad_val, jnp.int32),
            ]).reshape(2 * eh // K, K)

        src2 = halves(e[0, :e2], e[0, e2:] + roff, n_pad - 1)
        dscat2 = halves(e[1, :e2] - na + roff, e[1, e2:], TRASH)
        ddeg2 = halves(e[1, :e2] + roff, e[1, e2:], DEGPAD)
        return src2, dscat2, ddeg2

    prepped = tuple(_prep_edges(e, g) for e, g in zip(edges, GRAPHS))
    srcs = tuple(p[0] for p in prepped)
    dscats = tuple(p[1] for p in prepped)
    ddegs = tuple(p[2] for p in prepped)

    o_pt, o_pu, o_tu, o_po, o_uo = _sc_gcn(xs, srcs, dscats, ddegs)

    zu0, zu1, zu2 = o_pu[PAR:PAR + ITEM], o_tu[:ITEM], o_uo[:ITEM]
    zp0, zp1, zp2 = o_pu[:PAR], o_pt[:PAR], o_po[:PAR]

    pU = params["attn_U"]
    pP = params["attn_P"]
    wsU = _scores(jnp.concatenate([zu0, zu1, zu2]), pU["W"], pU["b"], pU["q"], ITEM)
    wsP = _scores(jnp.concatenate([zp0, zp1, zp2]), pP["W"], pP["b"], pP["q"], PAR)
    betaU = jax.nn.softmax(wsU)
    betaP = jax.nn.softmax(wsP)

    ti_wp = jnp.pad(params["ti_W"].T, ((0, 0), (0, DIM - TRAIT)))
    ti_bp = jnp.pad(params["ti_b"], (0, DIM - TRAIT)).reshape(1, DIM)
    tp_wp = jnp.pad(params["tp_W"].T, ((0, 0), (0, DIM - TRAIT)))
    tp_bp = jnp.pad(params["tp_b"], (0, DIM - TRAIT)).reshape(1, DIM)

    u_proj = _combine_project(zu0, zu1, zu2, betaU, ti_wp, ti_bp)
    p_proj = _combine_project(zp0, zp1, zp2, betaP, tp_wp, tp_bp)

    pg, ug, dg = _sc_batch_gather(p_proj, u_proj, params["disc_emb"][:, 0],
                                  par_id, item_id)

    traits_pad = jnp.pad(traits, ((0, 0), (0, DIM - TRAIT)))
    w1p = jnp.pad(params["p1_W"], ((0, 0), (0, DIM - TRAIT)))
    b1 = params["p1_b"].reshape(1, HID)
    w2 = params["p2_W"]
    b2 = params["p2_b"].reshape(1, HID // 2)
    w3p = jnp.pad(params["p3_W"], ((0, 128 - OPT), (0, 0)))
    b3p = jnp.pad(params["p3_b"], (0, 128 - OPT)).reshape(1, 128)

    out = _mlp(pg, ug, dg.reshape(B, 1), traits_pad, w1p, b1, w2, b2, w3p, b3p)
    return out[:, :OPT]
